# baseline (speedup 1.0000x reference)
import functools

import jax
import jax.numpy as jnp
from jax import lax
from jax.experimental import pallas as pl
from jax.experimental.pallas import tpu as pltpu
from jax.experimental.pallas import tpu_sc as plsc

START = 2000
NSUB = 8000
D = 128
NTOT = 10000
NE = 128000

NC = 2
NS = 16
NW = NC * NS
L = 16

RPW = 256
LAST_ROWS = NSUB - RPW * (NW - 1)
EPW = 4096
EROWS = EPW // 128
EPAD = NW * EPW - NE
DEGP = 8192
SHLEN = DEGP + D
ZLEN = SHLEN // NS

_mesh = plsc.VectorSubcoreMesh(core_axis_name="c", subcore_axis_name="s")
_params = pltpu.CompilerParams(use_tc_tiling_on_sc=False, needs_layout_passes=False)


@functools.partial(
    pl.kernel,
    out_type=(
        jax.ShapeDtypeStruct((NC, D), jnp.float32),
        jax.ShapeDtypeStruct((NC, DEGP), jnp.float32),
    ),
    mesh=_mesh,
    scratch_types=[
        pltpu.VMEM((RPW, D), jnp.float32),
        pltpu.VMEM((EROWS, 128), jnp.int32),
        pltpu.VMEM((128,), jnp.float32),
        pltpu.VMEM((128,), jnp.int32),
        pltpu.VMEM((D,), jnp.float32),
        pltpu.VMEM((528,), jnp.float32),
        pltpu.VMEM_SHARED((SHLEN,), jnp.float32),
        pltpu.SemaphoreType.DMA,
        pltpu.SemaphoreType.DMA,
    ],
    compiler_params=_params,
)
def _partials_kernel(emb, adjp, ps_out, deg_out,
                     chunk, eidx, ones, cidx, sbuf, zbuf, shacc, sem1, sem2):
    c = lax.axis_index("c")
    s = lax.axis_index("s")
    w = c * NS + s

    edma = pltpu.async_copy(adjp.at[w], eidx, sem1)

    @pl.when(w < NW - 1)
    def _():
        pltpu.sync_copy(emb.at[pl.ds(START + RPW * w, RPW)], chunk)

    @pl.when(w == NW - 1)
    def _():
        pltpu.sync_copy(emb.at[pl.ds(START + RPW * (NW - 1), LAST_ROWS)],
                        chunk.at[pl.ds(0, LAST_ROWS)])

    zero16 = jnp.zeros((L,), jnp.float32)
    one16 = jnp.full((L,), 1.0, jnp.float32)
    for k in range(528 // L):
        zbuf[pl.ds(L * k, L)] = zero16
    for k in range(128 // L):
        ones[pl.ds(L * k, L)] = one16
        cidx[pl.ds(L * k, L)] = lax.iota(jnp.int32, L) + (DEGP + L * k)
    pltpu.sync_copy(zbuf.at[pl.ds(0, ZLEN)], shacc.at[pl.ds(s * ZLEN, ZLEN)])

    nr = jnp.where(w < NW - 1, RPW, LAST_ROWS)

    def body(r, acc):
        return tuple(acc[i] + chunk[r, pl.ds(L * i, L)] for i in range(D // L))

    acc = lax.fori_loop(0, nr, body,
                        tuple(jnp.zeros((L,), jnp.float32) for _ in range(D // L)))
    for i in range(D // L):
        sbuf[pl.ds(L * i, L)] = acc[i]

    edma.wait()
    plsc.subcore_barrier()

    descs = [pltpu.async_copy(ones, shacc.at[eidx.at[j]], sem2, add=True)
             for j in range(EROWS)]
    descs.append(pltpu.async_copy(sbuf, shacc.at[cidx], sem2, add=True))
    for d_ in descs:
        d_.wait()
    plsc.subcore_barrier()

    @pl.when(s == 0)
    def _():
        pltpu.sync_copy(shacc.at[pl.ds(0, DEGP)], deg_out.at[c])
        pltpu.sync_copy(shacc.at[pl.ds(DEGP, D)], ps_out.at[c])


@functools.partial(
    pl.kernel,
    out_type=jax.ShapeDtypeStruct((NTOT, D), jnp.float32),
    mesh=_mesh,
    scratch_types=[
        pltpu.VMEM((RPW, D), jnp.float32),
        pltpu.VMEM((NC, D), jnp.float32),
        pltpu.VMEM((NC, RPW), jnp.float32),
        pltpu.VMEM((RPW,), jnp.float32),
    ],
    compiler_params=_params,
)
def _update_kernel(emb, ps2, deg2, out, chunk, psv, dgv, tbuf):
    c = lax.axis_index("c")
    s = lax.axis_index("s")
    w = c * NS + s

    @pl.when(w < 16)
    def _():
        pltpu.sync_copy(emb.at[pl.ds(125 * w, 125)], chunk.at[pl.ds(0, 125)])
        pltpu.sync_copy(chunk.at[pl.ds(0, 125)], out.at[pl.ds(125 * w, 125)])

    pltpu.sync_copy(ps2, psv)
    S = [psv[0, pl.ds(L * i, L)] + psv[1, pl.ds(L * i, L)] for i in range(D // L)]

    pltpu.sync_copy(deg2.at[0, pl.ds(RPW * w, RPW)], dgv.at[0])
    pltpu.sync_copy(deg2.at[1, pl.ds(RPW * w, RPW)], dgv.at[1])
    for i in range(RPW // L):
        dsum = dgv[0, pl.ds(L * i, L)] + dgv[1, pl.ds(L * i, L)]
        tbuf[pl.ds(L * i, L)] = 1.0 / (1.0 + dsum)

    @pl.when(w < NW - 1)
    def _():
        pltpu.sync_copy(emb.at[pl.ds(START + RPW * w, RPW)], chunk)

    @pl.when(w == NW - 1)
    def _():
        pltpu.sync_copy(emb.at[pl.ds(START + RPW * (NW - 1), LAST_ROWS)],
                        chunk.at[pl.ds(0, LAST_ROWS)])

    def body(r, carry):
        tr = plsc.load_gather(tbuf, [jnp.full((L,), 0, jnp.int32) + r])
        for i in range(D // L):
            x = chunk[r, pl.ds(L * i, L)]
            chunk[r, pl.ds(L * i, L)] = (x + S[i]) * (1.0 - S[i] * tr)
        return carry

    lax.fori_loop(0, RPW, body, 0)

    @pl.when(w < NW - 1)
    def _():
        pltpu.sync_copy(chunk, out.at[pl.ds(START + RPW * w, RPW)])

    @pl.when(w == NW - 1)
    def _():
        pltpu.sync_copy(chunk.at[pl.ds(0, LAST_ROWS)],
                        out.at[pl.ds(START + RPW * (NW - 1), LAST_ROWS)])


def kernel(all_node_embedding, adj_nonzero_rows):
    adjp = jnp.concatenate(
        [adj_nonzero_rows.astype(jnp.int32),
         jnp.full((EPAD,), NSUB, jnp.int32)]).reshape(NW, EROWS, 128)
    ps2, deg2 = _partials_kernel(all_node_embedding, adjp)
    return _update_kernel(all_node_embedding, ps2, deg2)
